# Initial kernel scaffold; baseline (speedup 1.0000x reference)
import jax
import jax.numpy as jnp
from jax.experimental import pallas as pl


def kernel(token_ids, embedding):
    raise NotImplementedError("write your pallas kernel here")


# speedup vs baseline: 1.8301x; 1.8301x over previous
import functools

import jax
import jax.numpy as jnp
from jax import lax
from jax.experimental import pallas as pl
from jax.experimental.pallas import tpu as pltpu
from jax.experimental.pallas import tpu_sc as plsc

_D = 64
_B = 16384 * 50
_NC = 2
_NS = 16
_NW = _NC * _NS
_BPW = _B // _NW
_C = 512
_G = _BPW // _C

_mesh = plsc.VectorSubcoreMesh(core_axis_name="c", subcore_axis_name="s")


@functools.partial(
    pl.kernel,
    mesh=_mesh,
    compiler_params=pltpu.CompilerParams(use_tc_tiling_on_sc=False),
    out_type=jax.ShapeDtypeStruct((_B, _D), jnp.float32),
    scratch_types=[
        pltpu.VMEM((_BPW,), jnp.int32),
        pltpu.VMEM((_C, _D), jnp.float32),
        pltpu.SemaphoreType.DMA,
    ],
)
def _gather_all(idx_hbm, table_hbm, out_hbm, idx_v, rows_v, gsem):
    wid = lax.axis_index("s") * _NC + lax.axis_index("c")
    base = wid * _BPW
    pltpu.sync_copy(idx_hbm.at[pl.ds(base, _BPW)], idx_v)

    def body(g, carry):
        off = g * _C
        pltpu.async_copy(
            table_hbm.at[idx_v.at[pl.ds(off, _C)]], rows_v, gsem
        ).wait()
        pltpu.sync_copy(rows_v, out_hbm.at[pl.ds(base + off, _C)])
        return carry

    lax.fori_loop(0, _G, body, 0)


def kernel(token_ids, embedding):
    flat = token_ids.reshape(-1).astype(jnp.int32)
    out = _gather_all(flat, embedding)
    return out.reshape(token_ids.shape + (embedding.shape[1],))


# speedup vs baseline: 1.8742x; 1.0241x over previous
import functools

import jax
import jax.numpy as jnp
from jax import lax
from jax.experimental import pallas as pl
from jax.experimental.pallas import tpu as pltpu
from jax.experimental.pallas import tpu_sc as plsc

_D = 64
_B = 16384 * 50
_NC = 2
_NS = 16
_NW = _NC * _NS
_BPW = _B // _NW
_C = 256
_G = _BPW // _C
_NB = 4

_mesh = plsc.VectorSubcoreMesh(core_axis_name="c", subcore_axis_name="s")


@functools.partial(
    pl.kernel,
    mesh=_mesh,
    compiler_params=pltpu.CompilerParams(use_tc_tiling_on_sc=False),
    out_type=jax.ShapeDtypeStruct((_B, _D), jnp.float32),
    scratch_types=[
        pltpu.VMEM((_BPW,), jnp.int32),
        pltpu.VMEM((_NB, _C, _D), jnp.float32),
        pltpu.SemaphoreType.DMA((_NB,)),
        pltpu.SemaphoreType.DMA((_NB,)),
    ],
)
def _gather_all(idx_hbm, table_hbm, out_hbm, idx_v, rows_v, gsem, wsem):
    wid = lax.axis_index("s") * _NC + lax.axis_index("c")
    base = wid * _BPW
    pltpu.sync_copy(idx_hbm.at[pl.ds(base, _BPW)], idx_v)

    def g_start(g, b):
        pltpu.async_copy(
            table_hbm.at[idx_v.at[pl.ds(g * _C, _C)]], rows_v.at[b],
            gsem.at[b])

    def g_wait(b):
        pltpu.make_async_copy(
            table_hbm.at[idx_v.at[pl.ds(0, _C)]], rows_v.at[b],
            gsem.at[b]).wait()

    def w_start(g, b):
        pltpu.async_copy(
            rows_v.at[b], out_hbm.at[pl.ds(base + g * _C, _C)], wsem.at[b])

    def w_wait(b):
        pltpu.make_async_copy(
            rows_v.at[b], out_hbm.at[pl.ds(base, _C)], wsem.at[b]).wait()

    for b in range(_NB - 1):
        g_start(b, b)

    for j in range(_NB):
        g_wait(j)
        w_start(j, j)
        if j >= 1:
            w_wait(j - 1)
        g_start(j + _NB - 1, (j - 1) % _NB)

    def body(i, carry):
        for j in range(_NB):
            g = i * _NB + j
            g_wait(j)
            w_start(g, j)
            w_wait((j - 1) % _NB)
            g_start(g + _NB - 1, (j - 1) % _NB)
        return carry

    lax.fori_loop(1, _G // _NB - 1, body, 0)

    for j in range(_NB):
        g = _G - _NB + j
        g_wait(j)
        w_start(g, j)
        if j == 0:
            w_wait(_NB - 1)
            g_start(_G - 1, _NB - 1)

    for j in range(_NB):
        w_wait(j)


def kernel(token_ids, embedding):
    flat = token_ids.reshape(-1).astype(jnp.int32)
    out = _gather_all(flat, embedding)
    return out.reshape(token_ids.shape + (embedding.shape[1],))
